# baseline (speedup 1.0000x reference)
import functools

import jax
import jax.numpy as jnp
from jax import lax
from jax.experimental import pallas as pl
from jax.experimental.pallas import tpu as pltpu
from jax.experimental.pallas import tpu_sc as plsc

VOCAB = 100000
EMB = 128
MAX_LEN = 2048
BATCH = 4

_NC = 2
_NS = 16
_NW = _NC * _NS
_N = BATCH * MAX_LEN
_BPW = _N // _NW
_ICHUNK = 128
_NJ = _BPW // _ICHUNK

_mesh = plsc.VectorSubcoreMesh(core_axis_name="c", subcore_axis_name="s")


@functools.partial(
    pl.kernel,
    out_type=jax.ShapeDtypeStruct((_N, EMB), jnp.float32),
    mesh=_mesh,
    scratch_types=[
        pltpu.VMEM((_NJ, _ICHUNK), jnp.int32),
        pltpu.VMEM((_NJ, _ICHUNK), jnp.int32),
        pltpu.VMEM((_BPW, EMB), jnp.float32),
        pltpu.SemaphoreType.DMA,
    ],
)
def _emb_kernel(tok_hbm, idx_hbm, sid_hbm, seq_hbm, pos_hbm, out_hbm,
                idx_v, sid_v, acc_v, sem):
    wid = lax.axis_index("s") * _NC + lax.axis_index("c")
    base = wid * _BPW
    l0 = lax.rem(base, MAX_LEN)

    pltpu.sync_copy(idx_hbm.at[pl.ds(wid * _NJ, _NJ)], idx_v)
    pltpu.sync_copy(sid_hbm.at[pl.ds(wid * _NJ, _NJ)], sid_v)
    pltpu.sync_copy(pos_hbm.at[pl.ds(l0, _BPW)], acc_v)

    for j in range(_NJ):
        dst = acc_v.at[pl.ds(j * _ICHUNK, _ICHUNK)]
        pltpu.async_copy(tok_hbm.at[idx_v.at[j]], dst, sem, add=True).wait()
        pltpu.async_copy(seq_hbm.at[sid_v.at[j]], dst, sem, add=True).wait()

    pltpu.sync_copy(acc_v, out_hbm.at[pl.ds(base, _BPW)])


def kernel(inputs, sequence_id, token_table, seq_table, pos_table):
    idx = jnp.reshape(inputs.astype(jnp.int32), (_N // _ICHUNK, _ICHUNK))
    sid = jnp.reshape(sequence_id.astype(jnp.int32), (_N // _ICHUNK, _ICHUNK))
    out = _emb_kernel(token_table, idx, sid, seq_table, pos_table)
    return jnp.reshape(out, (BATCH, MAX_LEN, EMB))


# speedup vs baseline: 1.0101x; 1.0101x over previous
import functools

import jax
import jax.numpy as jnp
from jax import lax
from jax.experimental import pallas as pl
from jax.experimental.pallas import tpu as pltpu
from jax.experimental.pallas import tpu_sc as plsc

VOCAB = 100000
EMB = 128
MAX_LEN = 2048
BATCH = 4

_NC = 2
_NS = 16
_NW = _NC * _NS
_N = BATCH * MAX_LEN
_BPW = _N // _NW
_ICHUNK = 128
_NJ = _BPW // _ICHUNK

_mesh = plsc.VectorSubcoreMesh(core_axis_name="c", subcore_axis_name="s")


@functools.partial(
    pl.kernel,
    out_type=jax.ShapeDtypeStruct((_N, EMB), jnp.float32),
    mesh=_mesh,
    scratch_types=[
        pltpu.VMEM((_NJ, _ICHUNK), jnp.int32),
        pltpu.VMEM((_NJ, _ICHUNK), jnp.int32),
        pltpu.VMEM((_BPW, EMB), jnp.float32),
        pltpu.SemaphoreType.DMA,
    ],
)
def _emb_kernel(tok_hbm, idx_hbm, sid_hbm, seq_hbm, pos_hbm, out_hbm,
                idx_v, sid_v, acc_v, sem):
    wid = lax.axis_index("s") * _NC + lax.axis_index("c")
    base = wid * _BPW
    l0 = lax.rem(base, MAX_LEN)

    cps = [
        pltpu.async_copy(idx_hbm.at[pl.ds(wid * _NJ, _NJ)], idx_v, sem),
        pltpu.async_copy(sid_hbm.at[pl.ds(wid * _NJ, _NJ)], sid_v, sem),
        pltpu.async_copy(pos_hbm.at[pl.ds(l0, _BPW)], acc_v, sem),
    ]
    for cp in cps:
        cp.wait()

    cps = []
    for j in range(_NJ):
        dst = acc_v.at[pl.ds(j * _ICHUNK, _ICHUNK)]
        cps.append(pltpu.async_copy(tok_hbm.at[idx_v.at[j]], dst, sem, add=True))
        cps.append(pltpu.async_copy(seq_hbm.at[sid_v.at[j]], dst, sem, add=True))
    for cp in cps:
        cp.wait()

    pltpu.sync_copy(acc_v, out_hbm.at[pl.ds(base, _BPW)])


def kernel(inputs, sequence_id, token_table, seq_table, pos_table):
    idx = jnp.reshape(inputs.astype(jnp.int32), (_N // _ICHUNK, _ICHUNK))
    sid = jnp.reshape(sequence_id.astype(jnp.int32), (_N // _ICHUNK, _ICHUNK))
    out = _emb_kernel(token_table, idx, sid, seq_table, pos_table)
    return jnp.reshape(out, (BATCH, MAX_LEN, EMB))


# speedup vs baseline: 6.2221x; 6.1601x over previous
import functools

import jax
import jax.numpy as jnp
from jax import lax
from jax.experimental import pallas as pl
from jax.experimental.pallas import tpu as pltpu
from jax.experimental.pallas import tpu_sc as plsc

VOCAB = 100000
EMB = 128
MAX_LEN = 2048
BATCH = 4

_NC = 2
_NS = 16
_NW = _NC * _NS
_N = BATCH * MAX_LEN
_BPW = _N // _NW
_ICHUNK = 128
_NJ = _BPW // _ICHUNK
_SREP = 128

_mesh = plsc.VectorSubcoreMesh(core_axis_name="c", subcore_axis_name="s")


@functools.partial(
    pl.kernel,
    out_type=jax.ShapeDtypeStruct((_N, EMB), jnp.float32),
    mesh=_mesh,
    scratch_types=[
        pltpu.VMEM((_NJ, _ICHUNK), jnp.int32),
        pltpu.VMEM((_NJ, _ICHUNK), jnp.int32),
        pltpu.VMEM((_BPW, EMB), jnp.float32),
        pltpu.SemaphoreType.DMA,
    ],
)
def _emb_kernel(tok_hbm, idx_hbm, sid_hbm, seq_hbm, pos_hbm, out_hbm,
                idx_v, sid_v, acc_v, sem):
    wid = lax.axis_index("s") * _NC + lax.axis_index("c")
    base = wid * _BPW
    l0 = lax.rem(base, MAX_LEN)

    cps = [
        pltpu.async_copy(idx_hbm.at[pl.ds(wid * _NJ, _NJ)], idx_v, sem),
        pltpu.async_copy(sid_hbm.at[pl.ds(wid * _NJ, _NJ)], sid_v, sem),
        pltpu.async_copy(pos_hbm.at[pl.ds(l0, _BPW)], acc_v, sem),
    ]
    for cp in cps:
        cp.wait()

    cps = []
    for j in range(_NJ):
        dst = acc_v.at[pl.ds(j * _ICHUNK, _ICHUNK)]
        cps.append(pltpu.async_copy(tok_hbm.at[idx_v.at[j]], dst, sem, add=True))
        cps.append(pltpu.async_copy(seq_hbm.at[sid_v.at[j]], dst, sem, add=True))
    for cp in cps:
        cp.wait()

    pltpu.sync_copy(acc_v, out_hbm.at[pl.ds(base, _BPW)])


def kernel(inputs, sequence_id, token_table, seq_table, pos_table):
    idx = jnp.reshape(inputs.astype(jnp.int32), (_N // _ICHUNK, _ICHUNK))
    salt = jax.lax.broadcasted_iota(jnp.int32, (_N // _ICHUNK, _ICHUNK), 1)
    sid = jnp.reshape(sequence_id.astype(jnp.int32), (_N // _ICHUNK, _ICHUNK))
    sid = sid + 2 * salt
    seq_rep = jnp.tile(seq_table, (_SREP, 1))
    out = _emb_kernel(token_table, idx, sid, seq_rep, pos_table)
    return jnp.reshape(out, (BATCH, MAX_LEN, EMB))
